# baseline (speedup 1.0000x reference)
import jax
import jax.numpy as jnp
from jax import lax
from jax.experimental import pallas as pl
from jax.experimental.pallas import tpu as pltpu
from jax.experimental.pallas import tpu_sc as plsc

_NC = 2
_NS = 16
_NW = _NC * _NS
_LANES = 16
_GCHUNK = 128


def _sc_gather(flat_idx, tab_flat):
    n, d = flat_idx.shape[0], tab_flat.shape[1]
    bpw = n // _NW
    nch = bpw // _GCHUNK
    b_per_table = n // 4

    mesh = plsc.VectorSubcoreMesh(core_axis_name="c", subcore_axis_name="s")

    @jax.jit
    def run(flat_idx, tab_flat):
        @pl.kernel(
            mesh=mesh,
            out_type=jax.ShapeDtypeStruct((n, d), jnp.float32),
            scratch_types=[
                pltpu.VMEM((nch, _GCHUNK), jnp.int32),
                pltpu.VMEM((bpw, d), jnp.float32),
                pltpu.SemaphoreType.DMA,
            ],
        )
        def k(idx_hbm, tab_hbm, out_hbm, idx_v, rows_v, sem):
            wid = lax.axis_index("s") * _NC + lax.axis_index("c")
            base = wid * bpw
            toff = (base // b_per_table) * (tab_hbm.shape[0] // 4)
            for j in range(nch):
                pltpu.sync_copy(
                    idx_hbm.at[pl.ds(base + j * _GCHUNK, _GCHUNK)], idx_v.at[j]
                )
            tvec = jnp.full((_LANES,), toff, dtype=jnp.int32)
            for j in range(nch):
                for i in range(_GCHUNK // _LANES):
                    sl = pl.ds(i * _LANES, _LANES)
                    idx_v[j, sl] = idx_v[j, sl] + tvec
            copies = []
            for j in range(nch):
                copies.append(
                    pltpu.make_async_copy(
                        tab_hbm.at[idx_v.at[j]],
                        rows_v.at[pl.ds(j * _GCHUNK, _GCHUNK)],
                        sem,
                    )
                )
                copies[-1].start()
            for c in copies:
                c.wait()
            pltpu.sync_copy(rows_v, out_hbm.at[pl.ds(base, bpw)])

        return k(flat_idx, tab_flat)

    return run(flat_idx, tab_flat)


_NBUF = 2
_NSPLIT = 4


def _make_concat_body(bB, nsteps):
    sb = bB // _NSPLIT

    def _out_copies(buf, out_hbm, sems, i, slot):
        return [
            pltpu.make_async_copy(
                buf.at[slot, pl.ds(q * sb, sb)],
                out_hbm.at[pl.ds(i * bB + q * sb, sb)],
                sems.at[slot, q],
            )
            for q in range(_NSPLIT)
        ]

    def _concat_body(emb_ref, *rest):
        feat_refs = rest[:_NSPLIT]
        out_hbm, buf, sems = rest[_NSPLIT], rest[_NSPLIT + 1], rest[_NSPLIT + 2]
        T = emb_ref.shape[0]
        i = pl.program_id(0)
        slot = lax.rem(i, _NBUF)

        @pl.when(i >= _NBUF)
        def _():
            for c in _out_copies(buf, out_hbm, sems, i - _NBUF, slot):
                c.wait()

        buf[slot, :, 0:T, :] = jnp.swapaxes(emb_ref[...], 0, 1)
        for q in range(_NSPLIT):
            buf[slot, pl.ds(q * sb, sb), T:, :] = feat_refs[q][...]
        for c in _out_copies(buf, out_hbm, sems, i, slot):
            c.start()

        @pl.when(i == nsteps - 1)
        def _():
            for k in range(1, _NBUF):
                j = i - _NBUF + k
                s = lax.rem(j, _NBUF)
                for c in _out_copies(buf, out_hbm, sems, j, s):
                    c.wait()
            for c in _out_copies(buf, out_hbm, sems, i, slot):
                c.wait()

    return _concat_body


def kernel(feature, indices, tables):
    B, L, D = feature.shape
    T, V, _ = tables.shape

    flat_idx = jnp.reshape(indices.astype(jnp.int32), (T * B,))
    tab_flat = jnp.reshape(tables, (T * V, D))

    embeds = _sc_gather(flat_idx, tab_flat)
    emb3 = jnp.reshape(embeds, (T, B, D))

    bB = 64
    sb = bB // _NSPLIT
    nsteps = B // bB

    def _feat_spec(q):
        return pl.BlockSpec((sb, L, D), lambda i, q=q: (i * _NSPLIT + q, 0, 0))

    out = pl.pallas_call(
        _make_concat_body(bB, nsteps),
        grid=(nsteps,),
        in_specs=[pl.BlockSpec((T, bB, D), lambda i: (0, i, 0))]
        + [_feat_spec(q) for q in range(_NSPLIT)],
        out_specs=pl.BlockSpec(memory_space=pl.ANY),
        out_shape=jax.ShapeDtypeStruct((B, L + T, D), jnp.float32),
        scratch_shapes=[
            pltpu.VMEM((_NBUF, bB, L + T, D), jnp.float32),
            pltpu.SemaphoreType.DMA((_NBUF, _NSPLIT)),
        ],
        compiler_params=pltpu.CompilerParams(
            dimension_semantics=("arbitrary",),
        ),
    )(emb3, *([feature] * _NSPLIT))
    return out


# speedup vs baseline: 1.8707x; 1.8707x over previous
import jax
import jax.numpy as jnp
from jax import lax
from jax.experimental import pallas as pl
from jax.experimental.pallas import tpu as pltpu
from jax.experimental.pallas import tpu_sc as plsc

_NC = 2
_NS = 16
_NW = _NC * _NS
_LANES = 16
_GCHUNK = 128


def _sc_gather(flat_idx, tab_flat):
    n, d = flat_idx.shape[0], tab_flat.shape[1]
    bpw = n // _NW
    nch = bpw // _GCHUNK
    b_per_table = n // 4

    mesh = plsc.VectorSubcoreMesh(core_axis_name="c", subcore_axis_name="s")

    @jax.jit
    def run(flat_idx, tab_flat):
        @pl.kernel(
            mesh=mesh,
            out_type=jax.ShapeDtypeStruct((n, d), jnp.float32),
            scratch_types=[
                pltpu.VMEM((nch, _GCHUNK), jnp.int32),
                pltpu.VMEM((bpw, d), jnp.float32),
                pltpu.SemaphoreType.DMA,
            ],
        )
        def k(idx_hbm, tab_hbm, out_hbm, idx_v, rows_v, sem):
            wid = lax.axis_index("s") * _NC + lax.axis_index("c")
            base = wid * bpw
            toff = (base // b_per_table) * (tab_hbm.shape[0] // 4)
            for j in range(nch):
                pltpu.sync_copy(
                    idx_hbm.at[pl.ds(base + j * _GCHUNK, _GCHUNK)], idx_v.at[j]
                )
            tvec = jnp.full((_LANES,), toff, dtype=jnp.int32)
            for j in range(nch):
                for i in range(_GCHUNK // _LANES):
                    sl = pl.ds(i * _LANES, _LANES)
                    idx_v[j, sl] = idx_v[j, sl] + tvec
            copies = []
            for j in range(nch):
                copies.append(
                    pltpu.make_async_copy(
                        tab_hbm.at[idx_v.at[j]],
                        rows_v.at[pl.ds(j * _GCHUNK, _GCHUNK)],
                        sem,
                    )
                )
                copies[-1].start()
            for c in copies:
                c.wait()
            pltpu.sync_copy(rows_v, out_hbm.at[pl.ds(base, bpw)])

        return k(flat_idx, tab_flat)

    return run(flat_idx, tab_flat)


_NBUF = 2


def _make_concat_body(bB, nsteps):
    def _out_copy(buf, out_hbm, sems, i, slot):
        return pltpu.make_async_copy(
            buf.at[slot], out_hbm.at[:, pl.ds(i * bB, bB), :], sems.at[slot]
        )

    def _concat_body(emb_ref, feat_ref, out_hbm, buf, sems):
        T = emb_ref.shape[0]
        i = pl.program_id(0)
        slot = lax.rem(i, _NBUF)

        @pl.when(i >= _NBUF)
        def _():
            _out_copy(buf, out_hbm, sems, i - _NBUF, slot).wait()

        buf[slot, 0:T] = emb_ref[...]
        buf[slot, T:] = jnp.swapaxes(feat_ref[...], 0, 1)
        _out_copy(buf, out_hbm, sems, i, slot).start()

        @pl.when(i == nsteps - 1)
        def _():
            for k in range(1, _NBUF):
                j = i - _NBUF + k
                _out_copy(buf, out_hbm, sems, j, lax.rem(j, _NBUF)).wait()
            _out_copy(buf, out_hbm, sems, i, slot).wait()

    return _concat_body


def kernel(feature, indices, tables):
    B, L, D = feature.shape
    T, V, _ = tables.shape

    flat_idx = jnp.reshape(indices.astype(jnp.int32), (T * B,))
    tab_flat = jnp.reshape(tables, (T * V, D))

    embeds = _sc_gather(flat_idx, tab_flat)
    emb3 = jnp.reshape(embeds, (T, B, D))

    bB = 64
    nsteps = B // bB

    out_t = pl.pallas_call(
        _make_concat_body(bB, nsteps),
        grid=(nsteps,),
        in_specs=[
            pl.BlockSpec((T, bB, D), lambda i: (0, i, 0)),
            pl.BlockSpec((bB, L, D), lambda i: (i, 0, 0)),
        ],
        out_specs=pl.BlockSpec(memory_space=pl.ANY),
        out_shape=jax.ShapeDtypeStruct((L + T, B, D), jnp.float32),
        scratch_shapes=[
            pltpu.VMEM((_NBUF, L + T, bB, D), jnp.float32),
            pltpu.SemaphoreType.DMA((_NBUF,)),
        ],
        compiler_params=pltpu.CompilerParams(
            dimension_semantics=("arbitrary",),
        ),
    )(emb3, feature)
    return jnp.transpose(out_t, (1, 0, 2))


# speedup vs baseline: 1.9150x; 1.0237x over previous
import jax
import jax.numpy as jnp
from jax import lax
from jax.experimental import pallas as pl
from jax.experimental.pallas import tpu as pltpu
from jax.experimental.pallas import tpu_sc as plsc

_NC = 2
_NS = 16
_NW = _NC * _NS
_LANES = 16
_GCHUNK = 128


def _sc_gather(flat_idx, tab_flat):
    n, d = flat_idx.shape[0], tab_flat.shape[1]
    bpw = n // _NW
    nch = bpw // _GCHUNK
    b_per_table = n // 4

    mesh = plsc.VectorSubcoreMesh(core_axis_name="c", subcore_axis_name="s")

    @jax.jit
    def run(flat_idx, tab_flat):
        @pl.kernel(
            mesh=mesh,
            out_type=jax.ShapeDtypeStruct((n, d), jnp.float32),
            scratch_types=[
                pltpu.VMEM((nch, _GCHUNK), jnp.int32),
                pltpu.VMEM((bpw, d), jnp.float32),
                pltpu.SemaphoreType.DMA,
            ],
        )
        def k(idx_hbm, tab_hbm, out_hbm, idx_v, rows_v, sem):
            wid = lax.axis_index("s") * _NC + lax.axis_index("c")
            base = wid * bpw
            toff = (base // b_per_table) * (tab_hbm.shape[0] // 4)
            for j in range(nch):
                pltpu.sync_copy(
                    idx_hbm.at[pl.ds(base + j * _GCHUNK, _GCHUNK)], idx_v.at[j]
                )
            tvec = jnp.full((_LANES,), toff, dtype=jnp.int32)
            for j in range(nch):
                for i in range(_GCHUNK // _LANES):
                    sl = pl.ds(i * _LANES, _LANES)
                    idx_v[j, sl] = idx_v[j, sl] + tvec
            copies = []
            for j in range(nch):
                copies.append(
                    pltpu.make_async_copy(
                        tab_hbm.at[idx_v.at[j]],
                        rows_v.at[pl.ds(j * _GCHUNK, _GCHUNK)],
                        sem,
                    )
                )
                copies[-1].start()
            for c in copies:
                c.wait()
            pltpu.sync_copy(rows_v, out_hbm.at[pl.ds(base, bpw)])

        return k(flat_idx, tab_flat)

    return run(flat_idx, tab_flat)


_NBUF = 2


def _make_concat_body(bB, nsteps):
    def _out_copy(buf, out_hbm, sems, i, slot):
        return pltpu.make_async_copy(
            buf.at[slot], out_hbm.at[:, pl.ds(i * bB, bB), :], sems.at[slot]
        )

    def _concat_body(emb_ref, feat_ref, out_hbm, buf, sems):
        T = emb_ref.shape[0]
        i = pl.program_id(0)
        slot = lax.rem(i, _NBUF)

        @pl.when(i >= _NBUF)
        def _():
            _out_copy(buf, out_hbm, sems, i - _NBUF, slot).wait()

        buf[slot, 0:T] = emb_ref[...]
        buf[slot, T:] = jnp.swapaxes(feat_ref[...], 0, 1)
        _out_copy(buf, out_hbm, sems, i, slot).start()

        @pl.when(i == nsteps - 1)
        def _():
            for k in range(1, _NBUF):
                j = i - _NBUF + k
                _out_copy(buf, out_hbm, sems, j, lax.rem(j, _NBUF)).wait()
            _out_copy(buf, out_hbm, sems, i, slot).wait()

    return _concat_body


def kernel(feature, indices, tables):
    B, L, D = feature.shape
    T, V, _ = tables.shape

    flat_idx = jnp.reshape(indices.astype(jnp.int32), (T * B,))
    tab_flat = jnp.reshape(tables, (T * V, D))

    embeds = _sc_gather(flat_idx, tab_flat)
    emb3 = jnp.reshape(embeds, (T, B, D))

    bB = 128
    nsteps = B // bB

    out_t = pl.pallas_call(
        _make_concat_body(bB, nsteps),
        grid=(nsteps,),
        in_specs=[
            pl.BlockSpec((T, bB, D), lambda i: (0, i, 0)),
            pl.BlockSpec((bB, L, D), lambda i: (i, 0, 0)),
        ],
        out_specs=pl.BlockSpec(memory_space=pl.ANY),
        out_shape=jax.ShapeDtypeStruct((L + T, B, D), jnp.float32),
        scratch_shapes=[
            pltpu.VMEM((_NBUF, L + T, bB, D), jnp.float32),
            pltpu.SemaphoreType.DMA((_NBUF,)),
        ],
        compiler_params=pltpu.CompilerParams(
            dimension_semantics=("arbitrary",),
        ),
    )(emb3, feature)
    return jnp.transpose(out_t, (1, 0, 2))
